# Initial kernel scaffold; baseline (speedup 1.0000x reference)
import functools

import jax
import jax.numpy as jnp
import numpy as np
from jax import lax
from jax.experimental import pallas as pl
from jax.experimental.pallas import tpu as pltpu
from jax.experimental.pallas import tpu_sc as plsc

_N_BITS = 8
_EPS = 1e-8
_THRESHOLD = 8.0

_NUM_CORES = 2
_NUM_SUBCORES = 16
_LANES = 16
_NW = _NUM_CORES * _NUM_SUBCORES

_CHUNK = 21504
_LUT_SIZE = 1024

_SCALE = np.float32(2.0 * (2.0 ** (_N_BITS - 1)) / (_THRESHOLD + _EPS))
_BIAS = np.float32(_LUT_SIZE / 2)


def _build_lut(cluster_centers):
    c = cluster_centers.reshape(-1).astype(jnp.float32)
    jj = jnp.arange(_LUT_SIZE, dtype=jnp.float32)
    t_rep = (jj - _BIAS) * 0.5 + 0.25
    t_rep = jnp.clip(t_rep, -(2.0 ** (_N_BITS - 1)), 2.0 ** (_N_BITS - 1) - 1)
    idx = jnp.argmin(jnp.abs(t_rep[:, None] - c[None, :]), axis=1)
    scale_back = np.float32(_THRESHOLD / (2.0 ** (_N_BITS - 1)))
    return jnp.take(c, idx) * scale_back


def _tile_body(x_hbm, lut_hbm, out_hbm, lut_v, ibuf, obuf, sem_in, sem_out,
               *, per_w, n_chunks):
    wid = lax.axis_index("c") * _NUM_SUBCORES + lax.axis_index("s")
    base = wid * per_w

    pltpu.sync_copy(lut_hbm, lut_v)

    def chunk_src(ci):
        return x_hbm.at[pl.ds(base + ci * _CHUNK, _CHUNK)]

    def chunk_dst(ci):
        return out_hbm.at[pl.ds(base + ci * _CHUNK, _CHUNK)]

    in_flight = [None] * n_chunks
    out_flight = [None] * n_chunks
    in_flight[0] = pltpu.async_copy(chunk_src(0), ibuf.at[0], sem_in)

    for ci in range(n_chunks):
        s = ci % 2
        if ci + 1 < n_chunks:
            in_flight[ci + 1] = pltpu.async_copy(
                chunk_src(ci + 1), ibuf.at[(ci + 1) % 2], sem_in)
        in_flight[ci].wait()
        if ci >= 2:
            out_flight[ci - 2].wait()

        ib = ibuf.at[s]
        ob = obuf.at[s]

        @plsc.parallel_loop(0, _CHUNK, step=_LANES, unroll=8)
        def _(i):
            xv = ib[pl.ds(i, _LANES)]
            u = xv * _SCALE + _BIAS
            u = jnp.minimum(jnp.maximum(u, 0.0), np.float32(_LUT_SIZE - 1))
            jv = u.astype(jnp.int32)
            ob[pl.ds(i, _LANES)] = plsc.load_gather(lut_v, [jv])

        out_flight[ci] = pltpu.async_copy(ob, chunk_dst(ci), sem_out)

    for ci in range(max(0, n_chunks - 2), n_chunks):
        out_flight[ci].wait()


@functools.partial(jax.jit, static_argnames=("n",))
def _run(x_flat, lut, n):
    per_w = n // _NW
    n_chunks = per_w // _CHUNK
    mesh = plsc.VectorSubcoreMesh(core_axis_name="c", subcore_axis_name="s")
    body = functools.partial(_tile_body, per_w=per_w, n_chunks=n_chunks)
    return pl.kernel(
        body,
        out_type=jax.ShapeDtypeStruct((n,), jnp.float32),
        mesh=mesh,
        scratch_types=[
            pltpu.VMEM((_LUT_SIZE,), jnp.float32),
            pltpu.VMEM((2, _CHUNK), jnp.float32),
            pltpu.VMEM((2, _CHUNK), jnp.float32),
            pltpu.SemaphoreType.DMA,
            pltpu.SemaphoreType.DMA,
        ],
    )(x_flat, lut)


def kernel(x, cluster_centers):
    orig_shape = x.shape
    n = x.size
    lut = _build_lut(cluster_centers)
    x_flat = x.reshape(-1)
    block = _NW * _CHUNK
    n_pad = -(-n // block) * block
    if n_pad != n:
        x_flat = jnp.pad(x_flat, (0, n_pad - n))
    out = _run(x_flat, lut, n_pad)
    if n_pad != n:
        out = out[:n]
    return out.reshape(orig_shape)


# speedup vs baseline: 325.0236x; 325.0236x over previous
import functools

import jax
import jax.numpy as jnp
import numpy as np
from jax import lax
from jax.experimental import pallas as pl
from jax.experimental.pallas import tpu as pltpu
from jax.experimental.pallas import tpu_sc as plsc

_N_BITS = 8
_EPS = 1e-8
_THRESHOLD = 8.0

_NUM_CORES = 2
_NUM_SUBCORES = 16
_LANES = 16
_NW = _NUM_CORES * _NUM_SUBCORES

_CHUNK = 21504
_LUT_SIZE = 1024

_SCALE = np.float32(2.0 * (2.0 ** (_N_BITS - 1)) / (_THRESHOLD + _EPS))
_BIAS = np.float32(_LUT_SIZE / 2)


def _build_lut(cluster_centers):
    c = cluster_centers.reshape(-1).astype(jnp.float32)
    jj = jnp.arange(_LUT_SIZE, dtype=jnp.float32)
    t_rep = (jj - _BIAS) * 0.5 + 0.25
    t_rep = jnp.clip(t_rep, -(2.0 ** (_N_BITS - 1)), 2.0 ** (_N_BITS - 1) - 1)
    idx = jnp.argmin(jnp.abs(t_rep[:, None] - c[None, :]), axis=1)
    scale_back = np.float32(_THRESHOLD / (2.0 ** (_N_BITS - 1)))
    return jnp.take(c, idx) * scale_back


def _tile_body(x_hbm, lut_hbm, out_hbm, lut_v, ibuf0, ibuf1, obuf0, obuf1,
               sem_in0, sem_in1, sem_out0, sem_out1, *, per_w, n_chunks):
    wid = lax.axis_index("c") * _NUM_SUBCORES + lax.axis_index("s")
    base = wid * per_w

    pltpu.sync_copy(lut_hbm, lut_v)

    ibufs = (ibuf0, ibuf1)
    obufs = (obuf0, obuf1)
    sems_in = (sem_in0, sem_in1)
    sems_out = (sem_out0, sem_out1)

    def chunk_src(ci):
        return x_hbm.at[pl.ds(base + ci * _CHUNK, _CHUNK)]

    def chunk_dst(ci):
        return out_hbm.at[pl.ds(base + ci * _CHUNK, _CHUNK)]

    in_flight = [None] * n_chunks
    out_flight = [None] * n_chunks
    in_flight[0] = pltpu.async_copy(chunk_src(0), ibufs[0], sems_in[0])

    for ci in range(n_chunks):
        s = ci % 2
        if ci + 1 < n_chunks:
            in_flight[ci + 1] = pltpu.async_copy(
                chunk_src(ci + 1), ibufs[(ci + 1) % 2], sems_in[(ci + 1) % 2])
        in_flight[ci].wait()
        if ci >= 2:
            out_flight[ci - 2].wait()

        ib = ibufs[s]
        ob = obufs[s]

        @plsc.parallel_loop(0, _CHUNK, step=_LANES, unroll=8)
        def _(i):
            xv = ib[pl.ds(i, _LANES)]
            u = xv * _SCALE + _BIAS
            u = jnp.minimum(jnp.maximum(u, 0.0), np.float32(_LUT_SIZE - 1))
            jv = u.astype(jnp.int32)
            ob[pl.ds(i, _LANES)] = plsc.load_gather(lut_v, [jv])

        out_flight[ci] = pltpu.async_copy(ob, chunk_dst(ci), sems_out[s])

    for ci in range(max(0, n_chunks - 2), n_chunks):
        out_flight[ci].wait()


@functools.partial(jax.jit, static_argnames=("n",))
def _run(x_flat, lut, n):
    per_w = n // _NW
    n_chunks = per_w // _CHUNK
    mesh = plsc.VectorSubcoreMesh(core_axis_name="c", subcore_axis_name="s")
    body = functools.partial(_tile_body, per_w=per_w, n_chunks=n_chunks)
    return pl.kernel(
        body,
        out_type=jax.ShapeDtypeStruct((n,), jnp.float32),
        mesh=mesh,
        compiler_params=pltpu.CompilerParams(needs_layout_passes=False),
        scratch_types=[
            pltpu.VMEM((_LUT_SIZE,), jnp.float32),
            pltpu.VMEM((_CHUNK,), jnp.float32),
            pltpu.VMEM((_CHUNK,), jnp.float32),
            pltpu.VMEM((_CHUNK,), jnp.float32),
            pltpu.VMEM((_CHUNK,), jnp.float32),
            pltpu.SemaphoreType.DMA,
            pltpu.SemaphoreType.DMA,
            pltpu.SemaphoreType.DMA,
            pltpu.SemaphoreType.DMA,
        ],
    )(x_flat, lut)


def kernel(x, cluster_centers):
    orig_shape = x.shape
    n = x.size
    lut = _build_lut(cluster_centers)
    x_flat = x.reshape(-1)
    block = _NW * _CHUNK
    n_pad = -(-n // block) * block
    if n_pad != n:
        x_flat = jnp.pad(x_flat, (0, n_pad - n))
    out = _run(x_flat, lut, n_pad)
    if n_pad != n:
        out = out[:n]
    return out.reshape(orig_shape)
